# Initial kernel scaffold; baseline (speedup 1.0000x reference)
import jax
import jax.numpy as jnp
from jax.experimental import pallas as pl


def kernel(x, patch_indices):
    raise NotImplementedError("write your pallas kernel here")


# speedup vs baseline: 1.7121x; 1.7121x over previous
import functools

import jax
import jax.numpy as jnp
from jax import lax
from jax.experimental import pallas as pl
from jax.experimental.pallas import tpu as pltpu
from jax.experimental.pallas import tpu_sc as plsc

B = 4
T = 64
BT = B * T
P_HEMI = 320
V = 153
H = 40962
VT = 2 * H
PI_FLAT = P_HEMI * V
ROWS_TOTAL = 2 * PI_FLAT

NUM_TILES = 32
CHUNK = 96
NCHUNKS = ROWS_TOTAL // CHUNK
CHUNKS_PER_HEMI = PI_FLAT // CHUNK
LANES = 16

_mesh = plsc.VectorSubcoreMesh(core_axis_name="c", subcore_axis_name="s")


@functools.partial(
    pl.kernel,
    out_type=jax.ShapeDtypeStruct((ROWS_TOTAL, BT), jnp.float32),
    mesh=_mesh,
    scratch_types=[
        pltpu.VMEM((CHUNK,), jnp.int32),
        pltpu.VMEM((CHUNK, BT), jnp.float32),
        pltpu.SemaphoreType.DMA,
    ],
)
def _sc_gather(xg_hbm, pi_hbm, out_hbm, idx_v, buf, sem):
    wid = lax.axis_index("s") * 2 + lax.axis_index("c")
    nchunks_w = jnp.where(wid < NCHUNKS % NUM_TILES, NCHUNKS // NUM_TILES + 1,
                          NCHUNKS // NUM_TILES)

    def chunk_body(i, carry):
        c = wid + i * NUM_TILES
        j0 = jnp.where(c < CHUNKS_PER_HEMI, c, c - CHUNKS_PER_HEMI) * CHUNK
        off = jnp.where(c < CHUNKS_PER_HEMI, 0, H).astype(jnp.int32)
        pltpu.sync_copy(pi_hbm.at[pl.ds(j0, CHUNK)], idx_v)
        off_vec = jnp.broadcast_to(off, (LANES,))
        for k in range(CHUNK // LANES):
            sl = pl.ds(k * LANES, LANES)
            idx_v[sl] = idx_v[sl] + off_vec
        pltpu.async_copy(xg_hbm.at[idx_v], buf, sem).wait()
        pltpu.sync_copy(buf, out_hbm.at[pl.ds(c * CHUNK, CHUNK)])
        return carry

    lax.fori_loop(0, nchunks_w, chunk_body, 0)


_HB = 4096


def _repack_body(in_ref, out_ref):
    out_ref[...] = jnp.concatenate([in_ref[b].T for b in range(B)], axis=1)


def _stage_a(xt):
    return pl.pallas_call(
        _repack_body,
        grid=(pl.cdiv(VT, _HB),),
        in_specs=[pl.BlockSpec((B, T, _HB), lambda h: (0, 0, h))],
        out_specs=pl.BlockSpec((_HB, BT), lambda h: (h, 0)),
        out_shape=jax.ShapeDtypeStruct((VT, BT), jnp.float32),
    )(xt)


_PB = 16


def _unpack_body(in_ref, out_ref):
    data_t = in_ref[...].T
    for b in range(B):
        out_ref[b] = data_t[b * T:(b + 1) * T].reshape(T, _PB, V)


def _stage_c(g):
    return pl.pallas_call(
        _unpack_body,
        grid=((2 * P_HEMI) // _PB,),
        in_specs=[pl.BlockSpec((_PB * V, BT), lambda p: (p, 0))],
        out_specs=pl.BlockSpec((B, T, _PB, V), lambda p: (0, 0, p, 0)),
        out_shape=jax.ShapeDtypeStruct((B, T, 2 * P_HEMI, V), jnp.float32),
    )(g)


def kernel(x, patch_indices):
    xt = jnp.transpose(x, (0, 2, 1))
    xg = _stage_a(xt)
    g = _sc_gather(xg, patch_indices.reshape(PI_FLAT))
    return _stage_c(g)
